# baseline (speedup 1.0000x reference)
import functools

import jax
import jax.numpy as jnp
from jax import lax
from jax.experimental import pallas as pl
from jax.experimental.pallas import tpu as pltpu
from jax.experimental.pallas import tpu_sc as plsc

BATCH = 16384
EMB = 64
PLOT_DIM = 384


def _make_sc_gather_packed(B):
    info = plsc.get_sparse_core_info()
    NC, NS = info.num_cores, info.num_subcores
    NW = NC * NS
    b_per_w = B // NW
    mesh = plsc.VectorSubcoreMesh(core_axis_name="c", subcore_axis_name="s")

    @functools.partial(
        pl.kernel,
        mesh=mesh,
        out_type=jax.ShapeDtypeStruct((B, 2 * EMB), jnp.float32),
        scratch_types=[
            pltpu.VMEM((b_per_w,), jnp.int32),
            pltpu.VMEM((b_per_w,), jnp.int32),
            pltpu.VMEM((b_per_w, 2 * EMB), jnp.float32),
            pltpu.SemaphoreType.DMA,
        ],
    )
    def gather_kernel(tab, idx, out, idx_v, idx2_v, rows_v, sem):
        wid = lax.axis_index("s") * NC + lax.axis_index("c")
        base = wid * b_per_w
        pltpu.sync_copy(idx.at[pl.ds(base, b_per_w)], idx_v)
        for g in range(b_per_w // 16):
            sl = pl.ds(g * 16, 16)
            idx2_v[sl] = lax.shift_right_logical(idx_v[sl], 1)
        pltpu.async_copy(tab.at[idx2_v], rows_v, sem).wait()
        pltpu.sync_copy(rows_v, out.at[pl.ds(base, b_per_w)])

    return gather_kernel


def _mlp_body(up_ref, mp_ref, upar_ref, mpar_ref, p_ref,
              w1u_ref, w1m_ref, w1p_ref, b1_ref,
              w2_ref, b2_ref, w3_ref, b3_ref, w4_ref, b4_ref, o_ref):
    upk = up_ref[...]
    mpk = mp_ref[...]
    u = jnp.where(upar_ref[...] == 1, upk[:, EMB:], upk[:, :EMB])
    m = jnp.where(mpar_ref[...] == 1, mpk[:, EMB:], mpk[:, :EMB])
    x = (jnp.dot(u, w1u_ref[...], preferred_element_type=jnp.float32)
         + jnp.dot(m, w1m_ref[...], preferred_element_type=jnp.float32)
         + jnp.dot(p_ref[...], w1p_ref[...], preferred_element_type=jnp.float32)
         + b1_ref[...])
    x = jnp.maximum(x, 0.0)
    x = jnp.maximum(
        jnp.dot(x, w2_ref[...], preferred_element_type=jnp.float32) + b2_ref[...], 0.0)
    x = jnp.maximum(
        jnp.dot(x, w3_ref[...], preferred_element_type=jnp.float32) + b3_ref[...], 0.0)
    o_ref[...] = jnp.dot(x, w4_ref[...], preferred_element_type=jnp.float32) + b4_ref[...]


def _mlp(upk, mpk, upar, mpar, plot, W1u, W1m, W1p, b1, W2, b2, W3, b3, W4, b4,
         block_rows):
    B = upk.shape[0]
    grid = (B // block_rows,)

    def rows(i):
        return (i, 0)

    def whole(i):
        return (0, 0)

    return pl.pallas_call(
        _mlp_body,
        grid=grid,
        in_specs=[
            pl.BlockSpec((block_rows, 2 * EMB), rows),
            pl.BlockSpec((block_rows, 2 * EMB), rows),
            pl.BlockSpec((block_rows, 1), rows),
            pl.BlockSpec((block_rows, 1), rows),
            pl.BlockSpec((block_rows, PLOT_DIM), rows),
            pl.BlockSpec(W1u.shape, whole),
            pl.BlockSpec(W1m.shape, whole),
            pl.BlockSpec(W1p.shape, whole),
            pl.BlockSpec(b1.shape, whole),
            pl.BlockSpec(W2.shape, whole),
            pl.BlockSpec(b2.shape, whole),
            pl.BlockSpec(W3.shape, whole),
            pl.BlockSpec(b3.shape, whole),
            pl.BlockSpec(W4.shape, whole),
            pl.BlockSpec(b4.shape, whole),
        ],
        out_specs=pl.BlockSpec((block_rows, 1), rows),
        out_shape=jax.ShapeDtypeStruct((B, 1), jnp.float32),
    )(upk, mpk, upar, mpar, plot, W1u, W1m, W1p, b1, W2, b2, W3, b3, W4, b4)


def kernel(users, movies, plot_embeddings, user_table, movie_table,
           W1, b1, W2, b2, W3, b3, W4, b4):
    u32 = users.astype(jnp.int32)
    m32 = movies.astype(jnp.int32)
    upk_tab = user_table.reshape(user_table.shape[0] // 2, 2 * EMB)
    mpk_tab = movie_table.reshape(movie_table.shape[0] // 2, 2 * EMB)
    gather = _make_sc_gather_packed(BATCH)
    mrows = gather(mpk_tab, m32)
    urows = gather(upk_tab, u32)
    upar = (u32 & 1).reshape(BATCH, 1)
    mpar = (m32 & 1).reshape(BATCH, 1)
    W1u = W1[:EMB]
    W1m = W1[EMB:2 * EMB]
    W1p = W1[2 * EMB:]
    return _mlp(urows, mrows, upar, mpar, plot_embeddings,
                W1u, W1m, W1p, b1.reshape(1, -1),
                W2, b2.reshape(1, -1), W3, b3.reshape(1, -1),
                W4, b4.reshape(1, -1), block_rows=2048)


# speedup vs baseline: 1.9200x; 1.9200x over previous
import functools

import jax
import jax.numpy as jnp
from jax import lax
from jax.experimental import pallas as pl
from jax.experimental.pallas import tpu as pltpu
from jax.experimental.pallas import tpu_sc as plsc

BATCH = 16384
EMB = 64
PLOT_DIM = 384
WIN = 128


def _make_sc_gather(B):
    info = plsc.get_sparse_core_info()
    NC, NS = info.num_cores, info.num_subcores
    NW = NC * NS
    b_per_w = B // NW
    mesh = plsc.VectorSubcoreMesh(core_axis_name="c", subcore_axis_name="s")

    @functools.partial(
        pl.kernel,
        mesh=mesh,
        compiler_params=pltpu.CompilerParams(
            disable_bounds_checks=True, needs_layout_passes=False),
        out_type=(
            jax.ShapeDtypeStruct((B, 2 * EMB), jnp.float32),
            jax.ShapeDtypeStruct((B, 2 * EMB), jnp.float32),
        ),
        scratch_types=[
            pltpu.VMEM((b_per_w,), jnp.int32),
            pltpu.VMEM((b_per_w,), jnp.int32),
            pltpu.VMEM((EMB, WIN), jnp.float32),
            pltpu.VMEM((b_per_w, 2 * EMB), jnp.float32),
            pltpu.SemaphoreType.DMA,
        ],
    )
    def gather_kernel(utabT, us, uperm, mtabT, ms, mperm, uout, mout,
                      idx_v, perm_v, win_v, rows_v, sem):
        wid = lax.axis_index("s") * NC + lax.axis_index("c")
        base = wid * b_per_w
        iota = lax.iota(jnp.int32, 16)
        cvecs = [iota + q * 16 for q in range(4)]

        def gather_sorted(tabT, sidx, perm, out):
            pltpu.sync_copy(sidx.at[pl.ds(base, b_per_w)], idx_v)
            pltpu.sync_copy(perm.at[pl.ds(base, b_per_w)], perm_v)

            def body(g, cur_w):
                vec = idx_v[pl.ds(g * 16, 16)]
                for j in range(16):
                    i = g * 16 + j
                    r = vec[j]
                    w = lax.shift_right_logical(r, 7)

                    @pl.when(w != cur_w)
                    def _():
                        off = pl.multiple_of(w * WIN, WIN)
                        pltpu.sync_copy(tabT.at[:, pl.ds(off, WIN)], win_v)

                    cur_w = w
                    colv = jnp.full((16,), lax.bitwise_and(r, WIN - 1),
                                    jnp.int32)
                    for q in range(4):
                        vals = plsc.load_gather(win_v, [cvecs[q], colv])
                        rows_v[i, pl.ds(q * 16, 16)] = vals
                return cur_w

            lax.fori_loop(0, b_per_w // 16, body, jnp.int32(-1))
            pltpu.async_copy(rows_v, out.at[perm_v], sem).wait()

        gather_sorted(mtabT, ms, mperm, mout)
        gather_sorted(utabT, us, uperm, uout)

    return gather_kernel


def _mlp_body(u_ref, m_ref, p_ref, w1u_ref, w1m_ref, w1p_ref, b1_ref,
              w2_ref, b2_ref, w3_ref, b3_ref, w4_ref, b4_ref, o_ref):
    u = u_ref[...][:, :EMB]
    m = m_ref[...][:, :EMB]
    x = (jnp.dot(u, w1u_ref[...], preferred_element_type=jnp.float32)
         + jnp.dot(m, w1m_ref[...], preferred_element_type=jnp.float32)
         + jnp.dot(p_ref[...], w1p_ref[...], preferred_element_type=jnp.float32)
         + b1_ref[...])
    x = jnp.maximum(x, 0.0)
    x = jnp.maximum(
        jnp.dot(x, w2_ref[...], preferred_element_type=jnp.float32) + b2_ref[...], 0.0)
    x = jnp.maximum(
        jnp.dot(x, w3_ref[...], preferred_element_type=jnp.float32) + b3_ref[...], 0.0)
    o_ref[...] = jnp.dot(x, w4_ref[...], preferred_element_type=jnp.float32) + b4_ref[...]


def _mlp(urows, mrows, plot, W1u, W1m, W1p, b1, W2, b2, W3, b3, W4, b4,
         block_rows):
    B = urows.shape[0]
    grid = (B // block_rows,)

    def rows(i):
        return (i, 0)

    def whole(i):
        return (0, 0)

    return pl.pallas_call(
        _mlp_body,
        grid=grid,
        in_specs=[
            pl.BlockSpec((block_rows, 2 * EMB), rows),
            pl.BlockSpec((block_rows, 2 * EMB), rows),
            pl.BlockSpec((block_rows, PLOT_DIM), rows),
            pl.BlockSpec(W1u.shape, whole),
            pl.BlockSpec(W1m.shape, whole),
            pl.BlockSpec(W1p.shape, whole),
            pl.BlockSpec(b1.shape, whole),
            pl.BlockSpec(W2.shape, whole),
            pl.BlockSpec(b2.shape, whole),
            pl.BlockSpec(W3.shape, whole),
            pl.BlockSpec(b3.shape, whole),
            pl.BlockSpec(W4.shape, whole),
            pl.BlockSpec(b4.shape, whole),
        ],
        out_specs=pl.BlockSpec((block_rows, 1), rows),
        out_shape=jax.ShapeDtypeStruct((B, 1), jnp.float32),
    )(urows, mrows, plot, W1u, W1m, W1p, b1, W2, b2, W3, b3, W4, b4)


def kernel(users, movies, plot_embeddings, user_table, movie_table,
           W1, b1, W2, b2, W3, b3, W4, b4):
    u32 = users.astype(jnp.int32)
    m32 = movies.astype(jnp.int32)
    iota = lax.iota(jnp.int32, BATCH)
    us, uperm = lax.sort_key_val(u32, iota)
    ms, mperm = lax.sort_key_val(m32, iota)
    urows, mrows = _make_sc_gather(BATCH)(
        user_table.T, us, uperm, movie_table.T, ms, mperm)
    W1u = W1[:EMB]
    W1m = W1[EMB:2 * EMB]
    W1p = W1[2 * EMB:]
    return _mlp(urows, mrows, plot_embeddings,
                W1u, W1m, W1p, b1.reshape(1, -1),
                W2, b2.reshape(1, -1), W3, b3.reshape(1, -1),
                W4, b4.reshape(1, -1), block_rows=2048)


# speedup vs baseline: 2.4346x; 1.2680x over previous
import functools

import jax
import jax.numpy as jnp
from jax import lax
from jax.experimental import pallas as pl
from jax.experimental.pallas import tpu as pltpu
from jax.experimental.pallas import tpu_sc as plsc

BATCH = 16384
EMB = 64
PLOT_DIM = 384
WIN = 128


def _make_sc_gather(B):
    info = plsc.get_sparse_core_info()
    NC, NS = info.num_cores, info.num_subcores
    NW = NC * NS
    b_per_w = B // NW
    mesh = plsc.VectorSubcoreMesh(core_axis_name="c", subcore_axis_name="s")

    @functools.partial(
        pl.kernel,
        mesh=mesh,
        compiler_params=pltpu.CompilerParams(
            disable_bounds_checks=True, needs_layout_passes=False),
        out_type=(
            jax.ShapeDtypeStruct((B, 2 * EMB), jnp.float32),
            jax.ShapeDtypeStruct((B, 2 * EMB), jnp.float32),
        ),
        scratch_types=[
            pltpu.VMEM((b_per_w,), jnp.int32),
            pltpu.VMEM((b_per_w,), jnp.int32),
            pltpu.VMEM((EMB, WIN), jnp.float32),
            pltpu.VMEM((EMB, WIN), jnp.float32),
            pltpu.VMEM((EMB, WIN), jnp.float32),
            pltpu.VMEM((EMB, WIN), jnp.float32),
            pltpu.VMEM((b_per_w, 2 * EMB), jnp.float32),
            pltpu.SemaphoreType.DMA,
            pltpu.SemaphoreType.DMA,
            pltpu.SemaphoreType.DMA,
            pltpu.SemaphoreType.DMA,
            pltpu.SemaphoreType.DMA,
        ],
    )
    def gather_kernel(utabT, us, uperm, mtabT, ms, mperm, uout, mout,
                      idx_v, perm_v, win0, win1, win2, win3, rows_v,
                      sem0, sem1, sem2, sem3, osem):
        wid = lax.axis_index("s") * NC + lax.axis_index("c")
        base = wid * b_per_w
        iota = lax.iota(jnp.int32, 16)
        cvecs = [iota + q * 16 for q in range(4)]
        wins = [win0, win1, win2, win3]
        sems = [sem0, sem1, sem2, sem3]
        NWALK = 4
        wpw = b_per_w // NWALK

        def gather_sorted(tabT, sidx, perm, out):
            pltpu.sync_copy(sidx.at[pl.ds(base, b_per_w)], idx_v)
            pltpu.sync_copy(perm.at[pl.ds(base, b_per_w)], perm_v)

            def body(g, carry):
                vecs = [idx_v[pl.ds(s * wpw + g * 16, 16)]
                        for s in range(NWALK)]
                for j in range(16):
                    rs = [vecs[s][j] for s in range(NWALK)]
                    wn = [lax.shift_right_logical(r, 7) for r in rs]
                    for s in range(NWALK):
                        @pl.when(wn[s] != carry[s])
                        def _(s=s):
                            off = pl.multiple_of(wn[s] * WIN, WIN)
                            pltpu.async_copy(
                                tabT.at[:, pl.ds(off, WIN)], wins[s], sems[s])
                    for s in range(NWALK):
                        @pl.when(wn[s] != carry[s])
                        def _(s=s):
                            pltpu.make_async_copy(
                                tabT.at[:, pl.ds(0, WIN)], wins[s],
                                sems[s]).wait()
                        i = s * wpw + g * 16 + j
                        colv = jnp.full((16,), lax.bitwise_and(rs[s], WIN - 1),
                                        jnp.int32)
                        for q in range(4):
                            vals = plsc.load_gather(wins[s], [cvecs[q], colv])
                            rows_v[i, pl.ds(q * 16, 16)] = vals
                    carry = tuple(wn)
                return carry

            lax.fori_loop(0, wpw // 16, body, (jnp.int32(-1),) * NWALK)
            pltpu.async_copy(rows_v, out.at[perm_v], osem).wait()

        gather_sorted(mtabT, ms, mperm, mout)
        gather_sorted(utabT, us, uperm, uout)

    return gather_kernel


def _mlp_body(u_ref, m_ref, p_ref, w1u_ref, w1m_ref, w1p_ref, b1_ref,
              w2_ref, b2_ref, w3_ref, b3_ref, w4_ref, b4_ref, o_ref):
    u = u_ref[...][:, :EMB]
    m = m_ref[...][:, :EMB]
    x = (jnp.dot(u, w1u_ref[...], preferred_element_type=jnp.float32)
         + jnp.dot(m, w1m_ref[...], preferred_element_type=jnp.float32)
         + jnp.dot(p_ref[...], w1p_ref[...], preferred_element_type=jnp.float32)
         + b1_ref[...])
    x = jnp.maximum(x, 0.0)
    x = jnp.maximum(
        jnp.dot(x, w2_ref[...], preferred_element_type=jnp.float32) + b2_ref[...], 0.0)
    x = jnp.maximum(
        jnp.dot(x, w3_ref[...], preferred_element_type=jnp.float32) + b3_ref[...], 0.0)
    o_ref[...] = jnp.dot(x, w4_ref[...], preferred_element_type=jnp.float32) + b4_ref[...]


def _mlp(urows, mrows, plot, W1u, W1m, W1p, b1, W2, b2, W3, b3, W4, b4,
         block_rows):
    B = urows.shape[0]
    grid = (B // block_rows,)

    def rows(i):
        return (i, 0)

    def whole(i):
        return (0, 0)

    return pl.pallas_call(
        _mlp_body,
        grid=grid,
        in_specs=[
            pl.BlockSpec((block_rows, 2 * EMB), rows),
            pl.BlockSpec((block_rows, 2 * EMB), rows),
            pl.BlockSpec((block_rows, PLOT_DIM), rows),
            pl.BlockSpec(W1u.shape, whole),
            pl.BlockSpec(W1m.shape, whole),
            pl.BlockSpec(W1p.shape, whole),
            pl.BlockSpec(b1.shape, whole),
            pl.BlockSpec(W2.shape, whole),
            pl.BlockSpec(b2.shape, whole),
            pl.BlockSpec(W3.shape, whole),
            pl.BlockSpec(b3.shape, whole),
            pl.BlockSpec(W4.shape, whole),
            pl.BlockSpec(b4.shape, whole),
        ],
        out_specs=pl.BlockSpec((block_rows, 1), rows),
        out_shape=jax.ShapeDtypeStruct((B, 1), jnp.float32),
    )(urows, mrows, plot, W1u, W1m, W1p, b1, W2, b2, W3, b3, W4, b4)


def kernel(users, movies, plot_embeddings, user_table, movie_table,
           W1, b1, W2, b2, W3, b3, W4, b4):
    u32 = users.astype(jnp.int32)
    m32 = movies.astype(jnp.int32)
    iota = lax.iota(jnp.int32, BATCH)
    us, uperm = lax.sort_key_val(u32, iota)
    ms, mperm = lax.sort_key_val(m32, iota)
    urows, mrows = _make_sc_gather(BATCH)(
        user_table.T, us, uperm, movie_table.T, ms, mperm)
    W1u = W1[:EMB]
    W1m = W1[EMB:2 * EMB]
    W1p = W1[2 * EMB:]
    return _mlp(urows, mrows, plot_embeddings,
                W1u, W1m, W1p, b1.reshape(1, -1),
                W2, b2.reshape(1, -1), W3, b3.reshape(1, -1),
                W4, b4.reshape(1, -1), block_rows=2048)
